# Initial kernel scaffold; baseline (speedup 1.0000x reference)
import functools

import jax
import jax.numpy as jnp
from jax import lax
from jax.experimental import pallas as pl
from jax.experimental.pallas import tpu as pltpu
from jax.experimental.pallas import tpu_sc as plsc

N = 10000
E = 320000
NC = 2
NS = 16
EPT = E // NS
CHUNK = 80
NCHUNK = EPT // CHUNK
RPT = N // NS

_MESH = plsc.VectorSubcoreMesh(core_axis_name="c", subcore_axis_name="s")


def _make_agg(dh, with_deg, parity_edges):
    out_types = [jax.ShapeDtypeStruct((N, dh), jnp.float32),
                 jax.ShapeDtypeStruct((N, dh), jnp.float32)]
    scratch = [
        pltpu.VMEM((CHUNK,), jnp.int32),
        pltpu.VMEM((CHUNK,), jnp.int32),
        pltpu.VMEM((CHUNK, dh), jnp.float32),
        pltpu.VMEM_SHARED((N, dh), jnp.float32),
        pltpu.SemaphoreType.DMA,
    ]
    if with_deg:
        out_types += [jax.ShapeDtypeStruct((N, 16), jnp.float32),
                      jax.ShapeDtypeStruct((N, 16), jnp.float32)]
        scratch += [pltpu.VMEM((CHUNK, 16), jnp.float32),
                    pltpu.VMEM_SHARED((N, 16), jnp.float32)]

    @functools.partial(pl.kernel, out_type=tuple(out_types), mesh=_MESH,
                       scratch_types=scratch)
    def k(dataA, dataB, src_hbm, dst_hbm, zeros_hbm, ones_hbm, *refs):
        if with_deg:
            (accA_o, accB_o, degA_o, degB_o,
             srcv, dstv, rows, acc, sem, onesv, dacc) = refs
        else:
            accA_o, accB_o, srcv, dstv, rows, acc, sem = refs
        c = lax.axis_index("c")
        s = lax.axis_index("s")
        row0 = s * RPT

        pltpu.sync_copy(zeros_hbm.at[pl.ds(row0, RPT)],
                        acc.at[pl.ds(row0, RPT)])
        if with_deg:
            pltpu.sync_copy(ones_hbm.at[pl.ds(0, CHUNK)], onesv)
            pltpu.sync_copy(zeros_hbm.at[pl.ds(row0, RPT), pl.ds(0, 16)],
                            dacc.at[pl.ds(row0, RPT)])
        plsc.subcore_barrier()

        def run(data_hbm, ci):
            base = s * EPT

            @pl.loop(0, NCHUNK)
            def _(j):
                off = base + j * CHUNK

                def step():
                    pltpu.sync_copy(src_hbm.at[pl.ds(off, CHUNK)], srcv)
                    pltpu.sync_copy(dst_hbm.at[pl.ds(off, CHUNK)], dstv)
                    pltpu.async_copy(data_hbm.at[srcv], rows, sem).wait()
                    pltpu.sync_copy(rows, acc.at[dstv], add=True)
                    if with_deg:
                        @pl.when(j % 2 == ci)
                        def _():
                            pltpu.sync_copy(onesv, dacc.at[dstv], add=True)

                if parity_edges:
                    @pl.when(j % 2 == ci)
                    def _():
                        step()
                else:
                    step()

        @pl.when(c == 0)
        def _():
            run(dataA, 0)

        @pl.when(c == 1)
        def _():
            run(dataB, 1)

        plsc.subcore_barrier()

        @pl.when(c == 0)
        def _():
            pltpu.sync_copy(acc.at[pl.ds(row0, RPT)],
                            accA_o.at[pl.ds(row0, RPT)])
            if with_deg:
                pltpu.sync_copy(dacc.at[pl.ds(row0, RPT)],
                                degA_o.at[pl.ds(row0, RPT)])

        @pl.when(c == 1)
        def _():
            pltpu.sync_copy(acc.at[pl.ds(row0, RPT)],
                            accB_o.at[pl.ds(row0, RPT)])
            if with_deg:
                pltpu.sync_copy(dacc.at[pl.ds(row0, RPT)],
                                degB_o.at[pl.ds(row0, RPT)])

    return k


_agg64_deg = _make_agg(64, with_deg=True, parity_edges=False)
_agg128 = _make_agg(128, with_deg=False, parity_edges=False)
_agg16_par = _make_agg(16, with_deg=False, parity_edges=True)

_BN = 1000


def _dot(a, b):
    return jnp.dot(a, b, precision=lax.Precision.HIGHEST,
                   preferred_element_type=jnp.float32)


def _mm(x, w):
    k, m = w.shape

    def body(x_ref, w_ref, o_ref):
        o_ref[...] = _dot(x_ref[...], w_ref[...])

    return pl.pallas_call(
        body,
        grid=(N // _BN,),
        in_specs=[pl.BlockSpec((_BN, k), lambda i: (i, 0)),
                  pl.BlockSpec((k, m), lambda i: (0, 0))],
        out_specs=pl.BlockSpec((_BN, m), lambda i: (i, 0)),
        out_shape=jax.ShapeDtypeStruct((N, m), jnp.float32),
    )(x, w)


def _stage2(accA, accB, degA, degB, r1, wA, wB, w2l, b1):
    def body(aA, aB, dA, dB, r1_ref, wA_ref, wB_ref, w2l_ref, b1_ref,
             h1_o, t2A_o, t2B_o):
        deg = dA[:, 0:1] + dB[:, 0:1]
        r = 1.0 / jnp.maximum(deg, 1.0)
        pre = (_dot(aA[...] * r, wA_ref[...]) + _dot(aB[...] * r, wB_ref[...])
               + r1_ref[...] + b1_ref[...])
        h1 = jnp.maximum(pre, 0.0)
        h1_o[...] = h1
        t2 = _dot(h1, w2l_ref[...])
        t2A_o[...] = t2[:, :128]
        t2B_o[...] = t2[:, 128:]

    return pl.pallas_call(
        body,
        grid=(N // _BN,),
        in_specs=[pl.BlockSpec((_BN, 64), lambda i: (i, 0)),
                  pl.BlockSpec((_BN, 64), lambda i: (i, 0)),
                  pl.BlockSpec((_BN, 16), lambda i: (i, 0)),
                  pl.BlockSpec((_BN, 16), lambda i: (i, 0)),
                  pl.BlockSpec((_BN, 512), lambda i: (i, 0)),
                  pl.BlockSpec((64, 512), lambda i: (0, 0)),
                  pl.BlockSpec((64, 512), lambda i: (0, 0)),
                  pl.BlockSpec((512, 256), lambda i: (0, 0)),
                  pl.BlockSpec((1, 512), lambda i: (0, 0))],
        out_specs=[pl.BlockSpec((_BN, 512), lambda i: (i, 0)),
                   pl.BlockSpec((_BN, 128), lambda i: (i, 0)),
                   pl.BlockSpec((_BN, 128), lambda i: (i, 0))],
        out_shape=[jax.ShapeDtypeStruct((N, 512), jnp.float32),
                   jax.ShapeDtypeStruct((N, 128), jnp.float32),
                   jax.ShapeDtypeStruct((N, 128), jnp.float32)],
    )(accA, accB, degA, degB, r1, wA, wB, w2l, b1)


def _stage4(accA, accB, degA, degB, r2, w3l, b2):
    def body(aA, aB, dA, dB, r2_ref, w3l_ref, b2_ref, h2_o, t3_o):
        deg = dA[:, 0:1] + dB[:, 0:1]
        r = 1.0 / jnp.maximum(deg, 1.0)
        agg = jnp.concatenate([aA[...], aB[...]], axis=1)
        h2 = jnp.maximum(agg * r + r2_ref[...] + b2_ref[...], 0.0)
        h2_o[...] = h2
        t3_o[...] = _dot(h2, w3l_ref[...])

    return pl.pallas_call(
        body,
        grid=(N // _BN,),
        in_specs=[pl.BlockSpec((_BN, 128), lambda i: (i, 0)),
                  pl.BlockSpec((_BN, 128), lambda i: (i, 0)),
                  pl.BlockSpec((_BN, 16), lambda i: (i, 0)),
                  pl.BlockSpec((_BN, 16), lambda i: (i, 0)),
                  pl.BlockSpec((_BN, 256), lambda i: (i, 0)),
                  pl.BlockSpec((256, 16), lambda i: (0, 0)),
                  pl.BlockSpec((1, 256), lambda i: (0, 0))],
        out_specs=[pl.BlockSpec((_BN, 256), lambda i: (i, 0)),
                   pl.BlockSpec((_BN, 16), lambda i: (i, 0))],
        out_shape=[jax.ShapeDtypeStruct((N, 256), jnp.float32),
                   jax.ShapeDtypeStruct((N, 16), jnp.float32)],
    )(accA, accB, degA, degB, r2, w3l, b2)


def _stage6(accA, accB, degA, degB, r3, b3):
    def body(aA, aB, dA, dB, r3_ref, b3_ref, o_ref):
        deg = dA[:, 0:1] + dB[:, 0:1]
        r = 1.0 / jnp.maximum(deg, 1.0)
        o_ref[...] = (aA[...] + aB[...]) * r + r3_ref[...] + b3_ref[...]

    return pl.pallas_call(
        body,
        grid=(N // _BN,),
        in_specs=[pl.BlockSpec((_BN, 16), lambda i: (i, 0)),
                  pl.BlockSpec((_BN, 16), lambda i: (i, 0)),
                  pl.BlockSpec((_BN, 16), lambda i: (i, 0)),
                  pl.BlockSpec((_BN, 16), lambda i: (i, 0)),
                  pl.BlockSpec((_BN, 16), lambda i: (i, 0)),
                  pl.BlockSpec((1, 16), lambda i: (0, 0))],
        out_specs=pl.BlockSpec((_BN, 16), lambda i: (i, 0)),
        out_shape=jax.ShapeDtypeStruct((N, 16), jnp.float32),
    )(accA, accB, degA, degB, r3, b3)


def kernel(x, edge_index, W1_l, b1, W1_r, W2_l, b2, W2_r, W3_l, b3, W3_r):
    src = edge_index[0]
    dst = edge_index[1]

    xA = x[:, :64]
    xB = x[:, 64:]
    W1rT = W1_r.T
    W1lAT = W1_l[:, :64].T
    W1lBT = W1_l[:, 64:].T
    W2lT = W2_l.T
    W2rT = W2_r.T
    W3lTp = jnp.pad(W3_l, ((0, 9), (0, 0))).T
    W3rTp = jnp.pad(W3_r, ((0, 9), (0, 0))).T
    b1r = b1.reshape(1, 512)
    b2r = b2.reshape(1, 256)
    b3r = jnp.pad(b3, (0, 9)).reshape(1, 16)

    zeros64 = jnp.zeros((N, 64), jnp.float32)
    zeros128 = jnp.zeros((N, 128), jnp.float32)
    zeros16 = jnp.zeros((N, 16), jnp.float32)
    ones = jnp.ones((CHUNK, 16), jnp.float32)

    accA1, accB1, degA, degB = _agg64_deg(xA, xB, src, dst, zeros64, ones)
    r1 = _mm(x, W1rT)
    h1, t2A, t2B = _stage2(accA1, accB1, degA, degB, r1,
                           W1lAT, W1lBT, W2lT, b1r)

    accA2, accB2 = _agg128(t2A, t2B, src, dst, zeros128, ones)
    r2 = _mm(h1, W2rT)
    h2, t3 = _stage4(accA2, accB2, degA, degB, r2, W3lTp, b2r)

    pA3, pB3 = _agg16_par(t3, t3, src, dst, zeros16, ones)
    r3 = _mm(h2, W3rTp)
    outp = _stage6(pA3, pB3, degA, degB, r3, b3r)
    return outp[:, :7]


# speedup vs baseline: 5.8636x; 5.8636x over previous
import dataclasses
import functools

import jax
import jax.numpy as jnp
from jax import lax
from jax.experimental import pallas as pl
from jax.experimental.pallas import tpu as pltpu
from jax.experimental.pallas import tpu_sc as plsc

N = 10000
E = 320000
NC = 2
NS = 16
NW = NC * NS
CHUNK = 80
RPT = 624
TAIL = N - NS * RPT
HR = 80

_MESH = plsc.VectorSubcoreMesh(core_axis_name="c", subcore_axis_name="s")

_SC_PARAMS = pltpu.CompilerParams()
if "needs_layout_passes" in pltpu.CompilerParams.__dataclass_fields__:
    _SC_PARAMS = dataclasses.replace(_SC_PARAMS, needs_layout_passes=False)


def _copy_rows(s, src, dst):
    pltpu.sync_copy(src.at[pl.ds(s * RPT, RPT)], dst.at[pl.ds(s * RPT, RPT)])

    @pl.when(s == NS - 1)
    def _():
        pltpu.sync_copy(src.at[pl.ds(NS * RPT, TAIL)],
                        dst.at[pl.ds(NS * RPT, TAIL)])


def _make_agg(featsplit, with_deg):
    out_types = [jax.ShapeDtypeStruct((N, 128), jnp.float32),
                 jax.ShapeDtypeStruct((N, 128), jnp.float32)]
    scratch = [
        pltpu.VMEM((CHUNK,), jnp.int32),
        pltpu.VMEM((CHUNK,), jnp.int32),
        pltpu.VMEM((CHUNK, 128), jnp.float32),
        pltpu.VMEM_SHARED((N, 128), jnp.float32),
        pltpu.SemaphoreType.DMA,
    ]
    if with_deg:
        out_types.append(jax.ShapeDtypeStruct((NW * HR, 128), jnp.float32))
        scratch.append(pltpu.VMEM((HR, 128), jnp.float32))

    ept = E // NS if featsplit else E // NW
    nchunk = ept // CHUNK

    @functools.partial(pl.kernel, out_type=tuple(out_types), mesh=_MESH,
                       scratch_types=scratch, compiler_params=_SC_PARAMS)
    def k(dataA, dataB, src_hbm, dst_hbm, z128, *refs):
        if with_deg:
            accA_o, accB_o, hist_o, srcv, dstv, rows, acc, sem, hist = refs
        else:
            accA_o, accB_o, srcv, dstv, rows, acc, sem = refs
        c = lax.axis_index("c")
        s = lax.axis_index("s")
        w = c * NS + s

        _copy_rows(s, z128, acc)
        if with_deg:
            pltpu.sync_copy(z128.at[pl.ds(0, HR)], hist)
        plsc.subcore_barrier()

        def run(data_hbm):
            base = (s if featsplit else w) * ept

            @pl.loop(0, nchunk)
            def _(j):
                off = base + j * CHUNK
                pltpu.sync_copy(src_hbm.at[pl.ds(off, CHUNK)], srcv)
                pltpu.sync_copy(dst_hbm.at[pl.ds(off, CHUNK)], dstv)
                pltpu.async_copy(data_hbm.at[srcv], rows, sem).wait()
                pltpu.sync_copy(rows, acc.at[dstv], add=True)
                if with_deg:
                    one = jnp.full((16,), 1.0, jnp.float32)
                    for kk in range(CHUNK // 16):
                        d16 = dstv[pl.ds(kk * 16, 16)]
                        plsc.addupdate_scatter(
                            hist, [lax.shift_right_logical(d16, 7),
                                   lax.bitwise_and(d16, 127)], one)

        @pl.when(c == 0)
        def _():
            run(dataA)

        @pl.when(c == 1)
        def _():
            run(dataB)

        plsc.subcore_barrier()

        @pl.when(c == 0)
        def _():
            _copy_rows(s, acc, accA_o)

        @pl.when(c == 1)
        def _():
            _copy_rows(s, acc, accB_o)

        if with_deg:
            pltpu.sync_copy(hist, hist_o.at[pl.ds(w * HR, HR)])

    return k


_agg_split_deg = _make_agg(featsplit=False, with_deg=True)
_agg_feat = _make_agg(featsplit=True, with_deg=False)
_agg_split = _make_agg(featsplit=False, with_deg=False)

_BN = 1000


def _dot(a, b):
    return jnp.dot(a, b, precision=lax.Precision.HIGHEST,
                   preferred_element_type=jnp.float32)


def _mm(x, w):
    k, m = w.shape

    def body(x_ref, w_ref, o_ref):
        o_ref[...] = _dot(x_ref[...], w_ref[...])

    return pl.pallas_call(
        body,
        grid=(N // _BN,),
        in_specs=[pl.BlockSpec((_BN, k), lambda i: (i, 0)),
                  pl.BlockSpec((k, m), lambda i: (0, 0))],
        out_specs=pl.BlockSpec((_BN, m), lambda i: (i, 0)),
        out_shape=jax.ShapeDtypeStruct((N, m), jnp.float32),
    )(x, w)


def _stage2(accA, accB, recip, r1, w1l, w2l, b1):
    def body(aA, aB, rp, r1_ref, w1l_ref, w2l_ref, b1_ref,
             h1_o, t2A_o, t2B_o):
        mean1 = (aA[...] + aB[...]) * rp[...]
        pre = _dot(mean1, w1l_ref[...]) + r1_ref[...] + b1_ref[...]
        h1 = jnp.maximum(pre, 0.0)
        h1_o[...] = h1
        t2 = _dot(h1, w2l_ref[...])
        t2A_o[...] = t2[:, :128]
        t2B_o[...] = t2[:, 128:]

    return pl.pallas_call(
        body,
        grid=(N // _BN,),
        in_specs=[pl.BlockSpec((_BN, 128), lambda i: (i, 0)),
                  pl.BlockSpec((_BN, 128), lambda i: (i, 0)),
                  pl.BlockSpec((_BN, 1), lambda i: (i, 0)),
                  pl.BlockSpec((_BN, 512), lambda i: (i, 0)),
                  pl.BlockSpec((128, 512), lambda i: (0, 0)),
                  pl.BlockSpec((512, 256), lambda i: (0, 0)),
                  pl.BlockSpec((1, 512), lambda i: (0, 0))],
        out_specs=[pl.BlockSpec((_BN, 512), lambda i: (i, 0)),
                   pl.BlockSpec((_BN, 128), lambda i: (i, 0)),
                   pl.BlockSpec((_BN, 128), lambda i: (i, 0))],
        out_shape=[jax.ShapeDtypeStruct((N, 512), jnp.float32),
                   jax.ShapeDtypeStruct((N, 128), jnp.float32),
                   jax.ShapeDtypeStruct((N, 128), jnp.float32)],
    )(accA, accB, recip, r1, w1l, w2l, b1)


def _stage4(accA, accB, recip, r2, w3l, b2):
    def body(aA, aB, rp, r2_ref, w3l_ref, b2_ref, h2_o, t3_o):
        agg = jnp.concatenate([aA[...], aB[...]], axis=1)
        h2 = jnp.maximum(agg * rp[...] + r2_ref[...] + b2_ref[...], 0.0)
        h2_o[...] = h2
        t3_o[...] = _dot(h2, w3l_ref[...])

    return pl.pallas_call(
        body,
        grid=(N // _BN,),
        in_specs=[pl.BlockSpec((_BN, 128), lambda i: (i, 0)),
                  pl.BlockSpec((_BN, 128), lambda i: (i, 0)),
                  pl.BlockSpec((_BN, 1), lambda i: (i, 0)),
                  pl.BlockSpec((_BN, 256), lambda i: (i, 0)),
                  pl.BlockSpec((256, 128), lambda i: (0, 0)),
                  pl.BlockSpec((1, 256), lambda i: (0, 0))],
        out_specs=[pl.BlockSpec((_BN, 256), lambda i: (i, 0)),
                   pl.BlockSpec((_BN, 128), lambda i: (i, 0))],
        out_shape=[jax.ShapeDtypeStruct((N, 256), jnp.float32),
                   jax.ShapeDtypeStruct((N, 128), jnp.float32)],
    )(accA, accB, recip, r2, w3l, b2)


def _stage6(accA, accB, recip, r3, b3):
    def body(aA, aB, rp, r3_ref, b3_ref, o_ref):
        o_ref[...] = (aA[...] + aB[...]) * rp[...] + r3_ref[...] + b3_ref[...]

    return pl.pallas_call(
        body,
        grid=(N // _BN,),
        in_specs=[pl.BlockSpec((_BN, 128), lambda i: (i, 0)),
                  pl.BlockSpec((_BN, 128), lambda i: (i, 0)),
                  pl.BlockSpec((_BN, 1), lambda i: (i, 0)),
                  pl.BlockSpec((_BN, 128), lambda i: (i, 0)),
                  pl.BlockSpec((1, 128), lambda i: (0, 0))],
        out_specs=pl.BlockSpec((_BN, 128), lambda i: (i, 0)),
        out_shape=jax.ShapeDtypeStruct((N, 128), jnp.float32),
    )(accA, accB, recip, r3, b3)


def kernel(x, edge_index, W1_l, b1, W1_r, W2_l, b2, W2_r, W3_l, b3, W3_r):
    src = edge_index[0]
    dst = edge_index[1]

    W1rT = W1_r.T
    W1lT = W1_l.T
    W2lT = W2_l.T
    W2rT = W2_r.T
    W3lTp = jnp.pad(W3_l, ((0, 121), (0, 0))).T
    W3rTp = jnp.pad(W3_r, ((0, 121), (0, 0))).T
    b1r = b1.reshape(1, 512)
    b2r = b2.reshape(1, 256)
    b3r = jnp.pad(b3, (0, 121)).reshape(1, 128)

    z128 = jnp.zeros((N, 128), jnp.float32)

    pA1, pB1, hists = _agg_split_deg(x, x, src, dst, z128)
    r1 = _mm(x, W1rT)
    deg = hists.reshape(NW, HR * 128).sum(axis=0)[:N]
    recip = (1.0 / jnp.maximum(deg, 1.0)).reshape(N, 1)
    h1, t2A, t2B = _stage2(pA1, pB1, recip, r1, W1lT, W2lT, b1r)

    accA2, accB2 = _agg_feat(t2A, t2B, src, dst, z128)
    r2 = _mm(h1, W2rT)
    h2, t3 = _stage4(accA2, accB2, recip, r2, W3lTp, b2r)

    pA3, pB3 = _agg_split(t3, t3, src, dst, z128)
    r3 = _mm(h2, W3rTp)
    outp = _stage6(pA3, pB3, recip, r3, b3r)
    return outp[:, :7]


# speedup vs baseline: 13.5227x; 2.3062x over previous
import dataclasses
import functools

import jax
import jax.numpy as jnp
from jax import lax
from jax.experimental import pallas as pl
from jax.experimental.pallas import tpu as pltpu
from jax.experimental.pallas import tpu_sc as plsc

N = 10000
E = 320000
NC = 2
NS = 16
NW = NC * NS
CHUNK = 80
RPT = 624
TAIL = N - NS * RPT
HR = 80
DEPTH = 2

_MESH = plsc.VectorSubcoreMesh(core_axis_name="c", subcore_axis_name="s")

_SC_PARAMS = pltpu.CompilerParams()
if "needs_layout_passes" in pltpu.CompilerParams.__dataclass_fields__:
    _SC_PARAMS = dataclasses.replace(_SC_PARAMS, needs_layout_passes=False)


def _copy_rows(s, src, dst):
    pltpu.sync_copy(src.at[pl.ds(s * RPT, RPT)], dst.at[pl.ds(s * RPT, RPT)])

    @pl.when(s == NS - 1)
    def _():
        pltpu.sync_copy(src.at[pl.ds(NS * RPT, TAIL)],
                        dst.at[pl.ds(NS * RPT, TAIL)])


def _make_agg(featsplit, with_deg):
    out_types = [jax.ShapeDtypeStruct((N, 128), jnp.float32),
                 jax.ShapeDtypeStruct((N, 128), jnp.float32)]
    ept = E // NS if featsplit else E // NW
    nchunk = ept // CHUNK
    scratch = [
        pltpu.VMEM((ept,), jnp.int32),
        pltpu.VMEM((DEPTH, CHUNK), jnp.int32),
        pltpu.VMEM((DEPTH, CHUNK, 128), jnp.float32),
        pltpu.VMEM_SHARED((N, 128), jnp.float32),
    ] + [pltpu.SemaphoreType.DMA] * (2 * DEPTH + 1)
    if with_deg:
        out_types.append(jax.ShapeDtypeStruct((NW * HR, 128), jnp.float32))
        scratch.append(pltpu.VMEM((HR, 128), jnp.float32))

    @functools.partial(pl.kernel, out_type=tuple(out_types), mesh=_MESH,
                       scratch_types=scratch, compiler_params=_SC_PARAMS)
    def k(dataA, dataB, src_hbm, dst_hbm, z128, *refs):
        if with_deg:
            (accA_o, accB_o, hist_o, srcall, dstv, rows, acc,
             *sems, hist) = refs
        else:
            accA_o, accB_o, srcall, dstv, rows, acc, *sems = refs
        semG = sems[:DEPTH]
        semI = sems[DEPTH:2 * DEPTH]
        semP = sems[2 * DEPTH]
        c = lax.axis_index("c")
        s = lax.axis_index("s")
        w = c * NS + s
        base = (s if featsplit else w) * ept

        _copy_rows(s, z128, acc)
        pltpu.async_copy(src_hbm.at[pl.ds(base, ept)], srcall, semP)
        if with_deg:
            pltpu.sync_copy(z128.at[pl.ds(0, HR)], hist)
        plsc.subcore_barrier()
        pltpu.make_async_copy(src_hbm.at[pl.ds(base, ept)], srcall, semP
                              ).wait()

        def run(data_hbm):
            def issue(j, b):
                pltpu.async_copy(
                    data_hbm.at[srcall.at[pl.ds(j * CHUNK, CHUNK)]],
                    rows.at[b], semG[b])
                pltpu.async_copy(dst_hbm.at[pl.ds(base + j * CHUNK, CHUNK)],
                                 dstv.at[b], semI[b])

            def drain(j, b):
                pltpu.make_async_copy(
                    data_hbm.at[srcall.at[pl.ds(j * CHUNK, CHUNK)]],
                    rows.at[b], semG[b]).wait()
                pltpu.make_async_copy(
                    dst_hbm.at[pl.ds(base + j * CHUNK, CHUNK)],
                    dstv.at[b], semI[b]).wait()
                pltpu.sync_copy(rows.at[b], acc.at[dstv.at[b]], add=True)
                if with_deg:
                    one = jnp.full((16,), 1.0, jnp.float32)
                    for kk in range(CHUNK // 16):
                        d16 = dstv[b, pl.ds(kk * 16, 16)]
                        plsc.addupdate_scatter(
                            hist, [lax.shift_right_logical(d16, 7),
                                   lax.bitwise_and(d16, 127)], one)

            for b in range(DEPTH):
                issue(b, b)

            @pl.loop(0, nchunk // DEPTH)
            def _(jj):
                j0 = jj * DEPTH
                for b in range(DEPTH):
                    j = j0 + b
                    drain(j, b)
                    jn = j + DEPTH

                    @pl.when(jn < nchunk)
                    def _():
                        issue(jn, b)

            for b in range(nchunk % DEPTH):
                drain(nchunk - nchunk % DEPTH + b, b)

        @pl.when(c == 0)
        def _():
            run(dataA)

        @pl.when(c == 1)
        def _():
            run(dataB)

        plsc.subcore_barrier()

        @pl.when(c == 0)
        def _():
            _copy_rows(s, acc, accA_o)

        @pl.when(c == 1)
        def _():
            _copy_rows(s, acc, accB_o)

        if with_deg:
            pltpu.sync_copy(hist, hist_o.at[pl.ds(w * HR, HR)])

    return k


_agg_split_deg = _make_agg(featsplit=False, with_deg=True)
_agg_feat = _make_agg(featsplit=True, with_deg=False)
_agg_split = _make_agg(featsplit=False, with_deg=False)

_BN = 1000


def _dot(a, b):
    return jnp.dot(a, b, precision=lax.Precision.HIGHEST,
                   preferred_element_type=jnp.float32)


def _mm(x, w):
    k, m = w.shape

    def body(x_ref, w_ref, o_ref):
        o_ref[...] = _dot(x_ref[...], w_ref[...])

    return pl.pallas_call(
        body,
        grid=(N // _BN,),
        in_specs=[pl.BlockSpec((_BN, k), lambda i: (i, 0)),
                  pl.BlockSpec((k, m), lambda i: (0, 0))],
        out_specs=pl.BlockSpec((_BN, m), lambda i: (i, 0)),
        out_shape=jax.ShapeDtypeStruct((N, m), jnp.float32),
    )(x, w)


def _stage2(accA, accB, recip, r1, w1l, w2l, b1):
    def body(aA, aB, rp, r1_ref, w1l_ref, w2l_ref, b1_ref,
             h1_o, t2A_o, t2B_o):
        mean1 = (aA[...] + aB[...]) * rp[...]
        pre = _dot(mean1, w1l_ref[...]) + r1_ref[...] + b1_ref[...]
        h1 = jnp.maximum(pre, 0.0)
        h1_o[...] = h1
        t2 = _dot(h1, w2l_ref[...])
        t2A_o[...] = t2[:, :128]
        t2B_o[...] = t2[:, 128:]

    return pl.pallas_call(
        body,
        grid=(N // _BN,),
        in_specs=[pl.BlockSpec((_BN, 128), lambda i: (i, 0)),
                  pl.BlockSpec((_BN, 128), lambda i: (i, 0)),
                  pl.BlockSpec((_BN, 1), lambda i: (i, 0)),
                  pl.BlockSpec((_BN, 512), lambda i: (i, 0)),
                  pl.BlockSpec((128, 512), lambda i: (0, 0)),
                  pl.BlockSpec((512, 256), lambda i: (0, 0)),
                  pl.BlockSpec((1, 512), lambda i: (0, 0))],
        out_specs=[pl.BlockSpec((_BN, 512), lambda i: (i, 0)),
                   pl.BlockSpec((_BN, 128), lambda i: (i, 0)),
                   pl.BlockSpec((_BN, 128), lambda i: (i, 0))],
        out_shape=[jax.ShapeDtypeStruct((N, 512), jnp.float32),
                   jax.ShapeDtypeStruct((N, 128), jnp.float32),
                   jax.ShapeDtypeStruct((N, 128), jnp.float32)],
    )(accA, accB, recip, r1, w1l, w2l, b1)


def _stage4(accA, accB, recip, r2, w3l, b2):
    def body(aA, aB, rp, r2_ref, w3l_ref, b2_ref, h2_o, t3_o):
        agg = jnp.concatenate([aA[...], aB[...]], axis=1)
        h2 = jnp.maximum(agg * rp[...] + r2_ref[...] + b2_ref[...], 0.0)
        h2_o[...] = h2
        t3_o[...] = _dot(h2, w3l_ref[...])

    return pl.pallas_call(
        body,
        grid=(N // _BN,),
        in_specs=[pl.BlockSpec((_BN, 128), lambda i: (i, 0)),
                  pl.BlockSpec((_BN, 128), lambda i: (i, 0)),
                  pl.BlockSpec((_BN, 1), lambda i: (i, 0)),
                  pl.BlockSpec((_BN, 256), lambda i: (i, 0)),
                  pl.BlockSpec((256, 128), lambda i: (0, 0)),
                  pl.BlockSpec((1, 256), lambda i: (0, 0))],
        out_specs=[pl.BlockSpec((_BN, 256), lambda i: (i, 0)),
                   pl.BlockSpec((_BN, 128), lambda i: (i, 0))],
        out_shape=[jax.ShapeDtypeStruct((N, 256), jnp.float32),
                   jax.ShapeDtypeStruct((N, 128), jnp.float32)],
    )(accA, accB, recip, r2, w3l, b2)


def _stage6(accA, accB, recip, r3, b3):
    def body(aA, aB, rp, r3_ref, b3_ref, o_ref):
        o_ref[...] = (aA[...] + aB[...]) * rp[...] + r3_ref[...] + b3_ref[...]

    return pl.pallas_call(
        body,
        grid=(N // _BN,),
        in_specs=[pl.BlockSpec((_BN, 128), lambda i: (i, 0)),
                  pl.BlockSpec((_BN, 128), lambda i: (i, 0)),
                  pl.BlockSpec((_BN, 1), lambda i: (i, 0)),
                  pl.BlockSpec((_BN, 128), lambda i: (i, 0)),
                  pl.BlockSpec((1, 128), lambda i: (0, 0))],
        out_specs=pl.BlockSpec((_BN, 128), lambda i: (i, 0)),
        out_shape=jax.ShapeDtypeStruct((N, 128), jnp.float32),
    )(accA, accB, recip, r3, b3)


def kernel(x, edge_index, W1_l, b1, W1_r, W2_l, b2, W2_r, W3_l, b3, W3_r):
    src = edge_index[0]
    dst = edge_index[1]

    W1rT = W1_r.T
    W1lT = W1_l.T
    W2lT = W2_l.T
    W2rT = W2_r.T
    W3lTp = jnp.pad(W3_l, ((0, 121), (0, 0))).T
    W3rTp = jnp.pad(W3_r, ((0, 121), (0, 0))).T
    b1r = b1.reshape(1, 512)
    b2r = b2.reshape(1, 256)
    b3r = jnp.pad(b3, (0, 121)).reshape(1, 128)

    z128 = jnp.zeros((N, 128), jnp.float32)

    pA1, pB1, hists = _agg_split_deg(x, x, src, dst, z128)
    r1 = _mm(x, W1rT)
    deg = hists.reshape(NW, HR * 128).sum(axis=0)[:N]
    recip = (1.0 / jnp.maximum(deg, 1.0)).reshape(N, 1)
    h1, t2A, t2B = _stage2(pA1, pB1, recip, r1, W1lT, W2lT, b1r)

    accA2, accB2 = _agg_feat(t2A, t2B, src, dst, z128)
    r2 = _mm(h1, W2rT)
    h2, t3 = _stage4(accA2, accB2, recip, r2, W3lTp, b2r)

    pA3, pB3 = _agg_split(t3, t3, src, dst, z128)
    r3 = _mm(h2, W3rTp)
    outp = _stage6(pA3, pB3, recip, r3, b3r)
    return outp[:, :7]


# speedup vs baseline: 15.2546x; 1.1281x over previous
import dataclasses
import functools

import jax
import jax.numpy as jnp
from jax import lax
from jax.experimental import pallas as pl
from jax.experimental.pallas import tpu as pltpu
from jax.experimental.pallas import tpu_sc as plsc

N = 10000
E = 320000
NC = 2
NS = 16
NW = NC * NS
CHUNK = 80
RPT = 624
TAIL = N - NS * RPT
HR = 80

_MESH = plsc.VectorSubcoreMesh(core_axis_name="c", subcore_axis_name="s")

_SC_PARAMS = pltpu.CompilerParams()
if "needs_layout_passes" in pltpu.CompilerParams.__dataclass_fields__:
    _SC_PARAMS = dataclasses.replace(_SC_PARAMS, needs_layout_passes=False)


def _copy_rows(s, src, dst):
    pltpu.sync_copy(src.at[pl.ds(s * RPT, RPT)], dst.at[pl.ds(s * RPT, RPT)])

    @pl.when(s == NS - 1)
    def _():
        pltpu.sync_copy(src.at[pl.ds(NS * RPT, TAIL)],
                        dst.at[pl.ds(NS * RPT, TAIL)])


def _make_agg(featsplit, with_deg, depth):
    out_types = [jax.ShapeDtypeStruct((N, 128), jnp.float32),
                 jax.ShapeDtypeStruct((N, 128), jnp.float32)]
    ept = E // NS if featsplit else E // NW
    nchunk = ept // CHUNK
    scratch = [
        pltpu.VMEM((depth, CHUNK), jnp.int32),
        pltpu.VMEM((depth, CHUNK), jnp.int32),
        pltpu.VMEM((depth, CHUNK, 128), jnp.float32),
        pltpu.VMEM_SHARED((N, 128), jnp.float32),
    ] + [pltpu.SemaphoreType.DMA] * (3 * depth)
    if with_deg:
        out_types.append(jax.ShapeDtypeStruct((NW * HR, 128), jnp.float32))
        scratch.append(pltpu.VMEM((HR, 128), jnp.float32))

    @functools.partial(pl.kernel, out_type=tuple(out_types), mesh=_MESH,
                       scratch_types=scratch, compiler_params=_SC_PARAMS)
    def k(dataA, dataB, src_hbm, dst_hbm, z128, *refs):
        if with_deg:
            (accA_o, accB_o, hist_o, srcv, dstv, rows, acc,
             *sems, hist) = refs
        else:
            accA_o, accB_o, srcv, dstv, rows, acc, *sems = refs
        semG = sems[:depth]
        semI = sems[depth:2 * depth]
        semS = sems[2 * depth:3 * depth]
        c = lax.axis_index("c")
        s = lax.axis_index("s")
        w = c * NS + s
        base = (s if featsplit else w) * ept

        _copy_rows(s, z128, acc)
        if with_deg:
            pltpu.sync_copy(z128.at[pl.ds(0, HR)], hist)
        plsc.subcore_barrier()

        def run(data_hbm):
            def idx_issue(j, b):
                pltpu.async_copy(src_hbm.at[pl.ds(base + j * CHUNK, CHUNK)],
                                 srcv.at[b], semS[b])
                pltpu.async_copy(dst_hbm.at[pl.ds(base + j * CHUNK, CHUNK)],
                                 dstv.at[b], semI[b])

            def gather_launch(j, b):
                pltpu.make_async_copy(
                    src_hbm.at[pl.ds(base + j * CHUNK, CHUNK)],
                    srcv.at[b], semS[b]).wait()
                pltpu.async_copy(data_hbm.at[srcv.at[b]], rows.at[b],
                                 semG[b])

            def drain(j, b):
                pltpu.make_async_copy(data_hbm.at[srcv.at[b]], rows.at[b],
                                      semG[b]).wait()
                pltpu.make_async_copy(
                    dst_hbm.at[pl.ds(base + j * CHUNK, CHUNK)],
                    dstv.at[b], semI[b]).wait()
                pltpu.sync_copy(rows.at[b], acc.at[dstv.at[b]], add=True)
                if with_deg:
                    one = jnp.full((16,), 1.0, jnp.float32)
                    for kk in range(CHUNK // 16):
                        d16 = dstv[b, pl.ds(kk * 16, 16)]
                        plsc.addupdate_scatter(
                            hist, [lax.shift_right_logical(d16, 7),
                                   lax.bitwise_and(d16, 127)], one)

            for b in range(depth):
                idx_issue(b, b)
            for b in range(depth - 1):
                gather_launch(b, b)

            @pl.loop(0, nchunk // depth)
            def _(jj):
                j0 = jj * depth
                for b in range(depth):
                    j = j0 + b
                    drain(j, b)
                    jn = j + depth

                    @pl.when(jn < nchunk)
                    def _():
                        idx_issue(jn, b)

                    jg = j + depth - 1
                    bg = (b + depth - 1) % depth

                    @pl.when(jg < nchunk)
                    def _():
                        gather_launch(jg, bg)

            for b in range(nchunk % depth):
                drain(nchunk - nchunk % depth + b, b)

        @pl.when(c == 0)
        def _():
            run(dataA)

        @pl.when(c == 1)
        def _():
            run(dataB)

        plsc.subcore_barrier()

        @pl.when(c == 0)
        def _():
            _copy_rows(s, acc, accA_o)

        @pl.when(c == 1)
        def _():
            _copy_rows(s, acc, accB_o)

        if with_deg:
            pltpu.sync_copy(hist, hist_o.at[pl.ds(w * HR, HR)])

    return k


_agg_split_deg = _make_agg(featsplit=False, with_deg=True, depth=3)
_agg_feat = _make_agg(featsplit=True, with_deg=False, depth=4)
_agg_split = _make_agg(featsplit=False, with_deg=False, depth=4)

_BN = 1000


def _dot(a, b):
    return jnp.dot(a, b, precision=lax.Precision.HIGHEST,
                   preferred_element_type=jnp.float32)


def _mm(x, w):
    k, m = w.shape

    def body(x_ref, w_ref, o_ref):
        o_ref[...] = _dot(x_ref[...], w_ref[...])

    return pl.pallas_call(
        body,
        grid=(N // _BN,),
        in_specs=[pl.BlockSpec((_BN, k), lambda i: (i, 0)),
                  pl.BlockSpec((k, m), lambda i: (0, 0))],
        out_specs=pl.BlockSpec((_BN, m), lambda i: (i, 0)),
        out_shape=jax.ShapeDtypeStruct((N, m), jnp.float32),
    )(x, w)


def _stage2(accA, accB, recip, r1, w1l, w2l, b1):
    def body(aA, aB, rp, r1_ref, w1l_ref, w2l_ref, b1_ref,
             h1_o, t2A_o, t2B_o):
        mean1 = (aA[...] + aB[...]) * rp[...]
        pre = _dot(mean1, w1l_ref[...]) + r1_ref[...] + b1_ref[...]
        h1 = jnp.maximum(pre, 0.0)
        h1_o[...] = h1
        t2 = _dot(h1, w2l_ref[...])
        t2A_o[...] = t2[:, :128]
        t2B_o[...] = t2[:, 128:]

    return pl.pallas_call(
        body,
        grid=(N // _BN,),
        in_specs=[pl.BlockSpec((_BN, 128), lambda i: (i, 0)),
                  pl.BlockSpec((_BN, 128), lambda i: (i, 0)),
                  pl.BlockSpec((_BN, 1), lambda i: (i, 0)),
                  pl.BlockSpec((_BN, 512), lambda i: (i, 0)),
                  pl.BlockSpec((128, 512), lambda i: (0, 0)),
                  pl.BlockSpec((512, 256), lambda i: (0, 0)),
                  pl.BlockSpec((1, 512), lambda i: (0, 0))],
        out_specs=[pl.BlockSpec((_BN, 512), lambda i: (i, 0)),
                   pl.BlockSpec((_BN, 128), lambda i: (i, 0)),
                   pl.BlockSpec((_BN, 128), lambda i: (i, 0))],
        out_shape=[jax.ShapeDtypeStruct((N, 512), jnp.float32),
                   jax.ShapeDtypeStruct((N, 128), jnp.float32),
                   jax.ShapeDtypeStruct((N, 128), jnp.float32)],
    )(accA, accB, recip, r1, w1l, w2l, b1)


def _stage4(accA, accB, recip, r2, w3l, b2):
    def body(aA, aB, rp, r2_ref, w3l_ref, b2_ref, h2_o, t3_o):
        agg = jnp.concatenate([aA[...], aB[...]], axis=1)
        h2 = jnp.maximum(agg * rp[...] + r2_ref[...] + b2_ref[...], 0.0)
        h2_o[...] = h2
        t3_o[...] = _dot(h2, w3l_ref[...])

    return pl.pallas_call(
        body,
        grid=(N // _BN,),
        in_specs=[pl.BlockSpec((_BN, 128), lambda i: (i, 0)),
                  pl.BlockSpec((_BN, 128), lambda i: (i, 0)),
                  pl.BlockSpec((_BN, 1), lambda i: (i, 0)),
                  pl.BlockSpec((_BN, 256), lambda i: (i, 0)),
                  pl.BlockSpec((256, 128), lambda i: (0, 0)),
                  pl.BlockSpec((1, 256), lambda i: (0, 0))],
        out_specs=[pl.BlockSpec((_BN, 256), lambda i: (i, 0)),
                   pl.BlockSpec((_BN, 128), lambda i: (i, 0))],
        out_shape=[jax.ShapeDtypeStruct((N, 256), jnp.float32),
                   jax.ShapeDtypeStruct((N, 128), jnp.float32)],
    )(accA, accB, recip, r2, w3l, b2)


def _stage6(accA, accB, recip, r3, b3):
    def body(aA, aB, rp, r3_ref, b3_ref, o_ref):
        o_ref[...] = (aA[...] + aB[...]) * rp[...] + r3_ref[...] + b3_ref[...]

    return pl.pallas_call(
        body,
        grid=(N // _BN,),
        in_specs=[pl.BlockSpec((_BN, 128), lambda i: (i, 0)),
                  pl.BlockSpec((_BN, 128), lambda i: (i, 0)),
                  pl.BlockSpec((_BN, 1), lambda i: (i, 0)),
                  pl.BlockSpec((_BN, 128), lambda i: (i, 0)),
                  pl.BlockSpec((1, 128), lambda i: (0, 0))],
        out_specs=pl.BlockSpec((_BN, 128), lambda i: (i, 0)),
        out_shape=jax.ShapeDtypeStruct((N, 128), jnp.float32),
    )(accA, accB, recip, r3, b3)


def kernel(x, edge_index, W1_l, b1, W1_r, W2_l, b2, W2_r, W3_l, b3, W3_r):
    src = edge_index[0]
    dst = edge_index[1]

    W1rT = W1_r.T
    W1lT = W1_l.T
    W2lT = W2_l.T
    W2rT = W2_r.T
    W3lTp = jnp.pad(W3_l, ((0, 121), (0, 0))).T
    W3rTp = jnp.pad(W3_r, ((0, 121), (0, 0))).T
    b1r = b1.reshape(1, 512)
    b2r = b2.reshape(1, 256)
    b3r = jnp.pad(b3, (0, 121)).reshape(1, 128)

    z128 = jnp.zeros((N, 128), jnp.float32)

    pA1, pB1, hists = _agg_split_deg(x, x, src, dst, z128)
    r1 = _mm(x, W1rT)
    deg = hists.reshape(NW, HR * 128).sum(axis=0)[:N]
    recip = (1.0 / jnp.maximum(deg, 1.0)).reshape(N, 1)
    h1, t2A, t2B = _stage2(pA1, pB1, recip, r1, W1lT, W2lT, b1r)

    accA2, accB2 = _agg_feat(t2A, t2B, src, dst, z128)
    r2 = _mm(h1, W2rT)
    h2, t3 = _stage4(accA2, accB2, recip, r2, W3lTp, b2r)

    pA3, pB3 = _agg_split(t3, t3, src, dst, z128)
    r3 = _mm(h2, W3rTp)
    outp = _stage6(pA3, pB3, recip, r3, b3r)
    return outp[:, :7]


# speedup vs baseline: 16.0511x; 1.0522x over previous
import dataclasses
import functools

import jax
import jax.numpy as jnp
from jax import lax
from jax.experimental import pallas as pl
from jax.experimental.pallas import tpu as pltpu
from jax.experimental.pallas import tpu_sc as plsc

N = 10000
E = 320000
NC = 2
NS = 16
NW = NC * NS
CHUNK = 80
RPT = 624
TAIL = N - NS * RPT
HR = 80

_MESH = plsc.VectorSubcoreMesh(core_axis_name="c", subcore_axis_name="s")

_SC_PARAMS = pltpu.CompilerParams()
if "needs_layout_passes" in pltpu.CompilerParams.__dataclass_fields__:
    _SC_PARAMS = dataclasses.replace(_SC_PARAMS, needs_layout_passes=False)


def _copy_rows(s, src, dst):
    pltpu.sync_copy(src.at[pl.ds(s * RPT, RPT)], dst.at[pl.ds(s * RPT, RPT)])

    @pl.when(s == NS - 1)
    def _():
        pltpu.sync_copy(src.at[pl.ds(NS * RPT, TAIL)],
                        dst.at[pl.ds(NS * RPT, TAIL)])


def _make_agg(featsplit, with_deg, depth):
    out_types = [jax.ShapeDtypeStruct((N, 128), jnp.float32),
                 jax.ShapeDtypeStruct((N, 128), jnp.float32)]
    ept = E // NS if featsplit else E // NW
    nchunk = ept // CHUNK
    scratch = [
        pltpu.VMEM((depth, CHUNK), jnp.int32),
        pltpu.VMEM((depth, CHUNK), jnp.int32),
        pltpu.VMEM((depth, CHUNK, 128), jnp.float32),
        pltpu.VMEM_SHARED((N, 128), jnp.float32),
    ] + [pltpu.SemaphoreType.DMA] * (3 * depth)
    if with_deg:
        out_types.append(jax.ShapeDtypeStruct((NW, HR * 128), jnp.float32))
        scratch.append(pltpu.VMEM((HR * 128,), jnp.float32))

    @functools.partial(pl.kernel, out_type=tuple(out_types), mesh=_MESH,
                       scratch_types=scratch, compiler_params=_SC_PARAMS)
    def k(dataA, dataB, src_hbm, dst_hbm, z128, z1d_hbm, *refs):
        if with_deg:
            (accA_o, accB_o, hist_o, srcv, dstv, rows, acc,
             *sems, hist) = refs
        else:
            accA_o, accB_o, srcv, dstv, rows, acc, *sems = refs
        semG = sems[:depth]
        semI = sems[depth:2 * depth]
        semS = sems[2 * depth:3 * depth]
        c = lax.axis_index("c")
        s = lax.axis_index("s")
        w = c * NS + s
        base = (s if featsplit else w) * ept

        _copy_rows(s, z128, acc)
        if with_deg:
            pltpu.sync_copy(z1d_hbm, hist)
        plsc.subcore_barrier()

        def run(data_hbm):
            def idx_issue(j, b):
                pltpu.async_copy(src_hbm.at[pl.ds(base + j * CHUNK, CHUNK)],
                                 srcv.at[b], semS[b])
                pltpu.async_copy(dst_hbm.at[pl.ds(base + j * CHUNK, CHUNK)],
                                 dstv.at[b], semI[b])

            def gather_launch(j, b):
                pltpu.make_async_copy(
                    src_hbm.at[pl.ds(base + j * CHUNK, CHUNK)],
                    srcv.at[b], semS[b]).wait()
                pltpu.async_copy(data_hbm.at[srcv.at[b]], rows.at[b],
                                 semG[b])

            def drain(j, b):
                pltpu.make_async_copy(data_hbm.at[srcv.at[b]], rows.at[b],
                                      semG[b]).wait()
                pltpu.make_async_copy(
                    dst_hbm.at[pl.ds(base + j * CHUNK, CHUNK)],
                    dstv.at[b], semI[b]).wait()
                pltpu.sync_copy(rows.at[b], acc.at[dstv.at[b]], add=True)
                if with_deg:
                    one = jnp.full((16,), 1.0, jnp.float32)
                    for kk in range(CHUNK // 16):
                        d16 = dstv[b, pl.ds(kk * 16, 16)]
                        plsc.addupdate_scatter(hist, [d16], one)

            for b in range(depth):
                idx_issue(b, b)
            for b in range(depth - 1):
                gather_launch(b, b)

            @pl.loop(0, nchunk // depth)
            def _(jj):
                j0 = jj * depth
                for b in range(depth):
                    j = j0 + b
                    drain(j, b)
                    jn = j + depth

                    @pl.when(jn < nchunk)
                    def _():
                        idx_issue(jn, b)

                    jg = j + depth - 1
                    bg = (b + depth - 1) % depth

                    @pl.when(jg < nchunk)
                    def _():
                        gather_launch(jg, bg)

            for b in range(nchunk % depth):
                drain(nchunk - nchunk % depth + b, b)

        @pl.when(c == 0)
        def _():
            run(dataA)

        @pl.when(c == 1)
        def _():
            run(dataB)

        plsc.subcore_barrier()

        @pl.when(c == 0)
        def _():
            _copy_rows(s, acc, accA_o)

        @pl.when(c == 1)
        def _():
            _copy_rows(s, acc, accB_o)

        if with_deg:
            pltpu.sync_copy(hist, hist_o.at[w])

    return k


_agg_split_deg = _make_agg(featsplit=False, with_deg=True, depth=3)
_agg_feat = _make_agg(featsplit=True, with_deg=False, depth=4)
_agg_split = _make_agg(featsplit=False, with_deg=False, depth=4)

_BN = 1000


def _dot(a, b):
    return jnp.dot(a, b, preferred_element_type=jnp.float32)


def _mm(x, w):
    k, m = w.shape

    def body(x_ref, w_ref, o_ref):
        o_ref[...] = _dot(x_ref[...], w_ref[...])

    return pl.pallas_call(
        body,
        grid=(N // _BN,),
        in_specs=[pl.BlockSpec((_BN, k), lambda i: (i, 0)),
                  pl.BlockSpec((k, m), lambda i: (0, 0))],
        out_specs=pl.BlockSpec((_BN, m), lambda i: (i, 0)),
        out_shape=jax.ShapeDtypeStruct((N, m), jnp.float32),
    )(x, w)


def _stage2(accA, accB, recip, r1, w1l, w2l, b1):
    def body(aA, aB, rp, r1_ref, w1l_ref, w2l_ref, b1_ref,
             h1_o, t2A_o, t2B_o):
        mean1 = (aA[...] + aB[...]) * rp[...]
        pre = _dot(mean1, w1l_ref[...]) + r1_ref[...] + b1_ref[...]
        h1 = jnp.maximum(pre, 0.0)
        h1_o[...] = h1
        t2 = _dot(h1, w2l_ref[...])
        t2A_o[...] = t2[:, :128]
        t2B_o[...] = t2[:, 128:]

    return pl.pallas_call(
        body,
        grid=(N // _BN,),
        in_specs=[pl.BlockSpec((_BN, 128), lambda i: (i, 0)),
                  pl.BlockSpec((_BN, 128), lambda i: (i, 0)),
                  pl.BlockSpec((_BN, 1), lambda i: (i, 0)),
                  pl.BlockSpec((_BN, 512), lambda i: (i, 0)),
                  pl.BlockSpec((128, 512), lambda i: (0, 0)),
                  pl.BlockSpec((512, 256), lambda i: (0, 0)),
                  pl.BlockSpec((1, 512), lambda i: (0, 0))],
        out_specs=[pl.BlockSpec((_BN, 512), lambda i: (i, 0)),
                   pl.BlockSpec((_BN, 128), lambda i: (i, 0)),
                   pl.BlockSpec((_BN, 128), lambda i: (i, 0))],
        out_shape=[jax.ShapeDtypeStruct((N, 512), jnp.float32),
                   jax.ShapeDtypeStruct((N, 128), jnp.float32),
                   jax.ShapeDtypeStruct((N, 128), jnp.float32)],
    )(accA, accB, recip, r1, w1l, w2l, b1)


def _stage4(accA, accB, recip, r2, w3l, b2):
    def body(aA, aB, rp, r2_ref, w3l_ref, b2_ref, h2_o, t3_o):
        agg = jnp.concatenate([aA[...], aB[...]], axis=1)
        h2 = jnp.maximum(agg * rp[...] + r2_ref[...] + b2_ref[...], 0.0)
        h2_o[...] = h2
        t3_o[...] = _dot(h2, w3l_ref[...])

    return pl.pallas_call(
        body,
        grid=(N // _BN,),
        in_specs=[pl.BlockSpec((_BN, 128), lambda i: (i, 0)),
                  pl.BlockSpec((_BN, 128), lambda i: (i, 0)),
                  pl.BlockSpec((_BN, 1), lambda i: (i, 0)),
                  pl.BlockSpec((_BN, 256), lambda i: (i, 0)),
                  pl.BlockSpec((256, 128), lambda i: (0, 0)),
                  pl.BlockSpec((1, 256), lambda i: (0, 0))],
        out_specs=[pl.BlockSpec((_BN, 256), lambda i: (i, 0)),
                   pl.BlockSpec((_BN, 128), lambda i: (i, 0))],
        out_shape=[jax.ShapeDtypeStruct((N, 256), jnp.float32),
                   jax.ShapeDtypeStruct((N, 128), jnp.float32)],
    )(accA, accB, recip, r2, w3l, b2)


def _stage6(accA, accB, recip, r3, b3):
    def body(aA, aB, rp, r3_ref, b3_ref, o_ref):
        o_ref[...] = (aA[...] + aB[...]) * rp[...] + r3_ref[...] + b3_ref[...]

    return pl.pallas_call(
        body,
        grid=(N // _BN,),
        in_specs=[pl.BlockSpec((_BN, 128), lambda i: (i, 0)),
                  pl.BlockSpec((_BN, 128), lambda i: (i, 0)),
                  pl.BlockSpec((_BN, 1), lambda i: (i, 0)),
                  pl.BlockSpec((_BN, 128), lambda i: (i, 0)),
                  pl.BlockSpec((1, 128), lambda i: (0, 0))],
        out_specs=pl.BlockSpec((_BN, 128), lambda i: (i, 0)),
        out_shape=jax.ShapeDtypeStruct((N, 128), jnp.float32),
    )(accA, accB, recip, r3, b3)


def kernel(x, edge_index, W1_l, b1, W1_r, W2_l, b2, W2_r, W3_l, b3, W3_r):
    src = edge_index[0]
    dst = edge_index[1]

    W1rT = W1_r.T
    W1lT = W1_l.T
    W2lT = W2_l.T
    W2rT = W2_r.T
    W3lTp = jnp.pad(W3_l, ((0, 121), (0, 0))).T
    W3rTp = jnp.pad(W3_r, ((0, 121), (0, 0))).T
    b1r = b1.reshape(1, 512)
    b2r = b2.reshape(1, 256)
    b3r = jnp.pad(b3, (0, 121)).reshape(1, 128)

    z128 = jnp.zeros((N, 128), jnp.float32)
    z1d = jnp.zeros((HR * 128,), jnp.float32)

    pA1, pB1, hists = _agg_split_deg(x, x, src, dst, z128, z1d)
    r1 = _mm(x, W1rT)
    deg = hists.sum(axis=0)[:N]
    recip = (1.0 / jnp.maximum(deg, 1.0)).reshape(N, 1)
    h1, t2A, t2B = _stage2(pA1, pB1, recip, r1, W1lT, W2lT, b1r)

    accA2, accB2 = _agg_feat(t2A, t2B, src, dst, z128, z1d)
    r2 = _mm(h1, W2rT)
    h2, t3 = _stage4(accA2, accB2, recip, r2, W3lTp, b2r)

    pA3, pB3 = _agg_split(t3, t3, src, dst, z128, z1d)
    r3 = _mm(h2, W3rTp)
    outp = _stage6(pA3, pB3, recip, r3, b3r)
    return outp[:, :7]


# speedup vs baseline: 16.2705x; 1.0137x over previous
import dataclasses
import functools

import jax
import jax.numpy as jnp
from jax import lax
from jax.experimental import pallas as pl
from jax.experimental.pallas import tpu as pltpu
from jax.experimental.pallas import tpu_sc as plsc

N = 10000
E = 320000
NC = 2
NS = 16
NW = NC * NS
CHUNK = 80
RPT = 624
TAIL = N - NS * RPT
HR = 80

_MESH = plsc.VectorSubcoreMesh(core_axis_name="c", subcore_axis_name="s")

_SC_PARAMS = pltpu.CompilerParams()
if "needs_layout_passes" in pltpu.CompilerParams.__dataclass_fields__:
    _SC_PARAMS = dataclasses.replace(_SC_PARAMS, needs_layout_passes=False)


def _copy_rows(s, src, dst):
    pltpu.sync_copy(src.at[pl.ds(s * RPT, RPT)], dst.at[pl.ds(s * RPT, RPT)])

    @pl.when(s == NS - 1)
    def _():
        pltpu.sync_copy(src.at[pl.ds(NS * RPT, TAIL)],
                        dst.at[pl.ds(NS * RPT, TAIL)])


def _make_agg(featsplit, with_deg, depth):
    out_types = [jax.ShapeDtypeStruct((N, 128), jnp.float32),
                 jax.ShapeDtypeStruct((N, 128), jnp.float32)]
    ept = E // NS if featsplit else E // NW
    nchunk = ept // CHUNK
    scratch = [
        pltpu.VMEM((depth, CHUNK), jnp.int32),
        pltpu.VMEM((depth, CHUNK), jnp.int32),
        pltpu.VMEM((depth, CHUNK, 128), jnp.float32),
        pltpu.VMEM_SHARED((N, 128), jnp.float32),
    ] + [pltpu.SemaphoreType.DMA] * (3 * depth)
    if with_deg:
        out_types.append(jax.ShapeDtypeStruct((NW, HR * 128), jnp.float32))
        scratch.append(pltpu.VMEM((HR * 128,), jnp.float32))

    @functools.partial(pl.kernel, out_type=tuple(out_types), mesh=_MESH,
                       scratch_types=scratch, compiler_params=_SC_PARAMS)
    def k(dataA, dataB, src_hbm, dst_hbm, z128, z1d_hbm, *refs):
        if with_deg:
            (accA_o, accB_o, hist_o, srcv, dstv, rows, acc,
             *sems, hist) = refs
        else:
            accA_o, accB_o, srcv, dstv, rows, acc, *sems = refs
        semG = sems[:depth]
        semI = sems[depth:2 * depth]
        semS = sems[2 * depth:3 * depth]
        c = lax.axis_index("c")
        s = lax.axis_index("s")
        w = c * NS + s
        base = (s if featsplit else w) * ept

        _copy_rows(s, z128, acc)
        if with_deg:
            pltpu.sync_copy(z1d_hbm, hist)
        plsc.subcore_barrier()

        def run(data_hbm):
            def idx_issue(j, b):
                pltpu.async_copy(src_hbm.at[pl.ds(base + j * CHUNK, CHUNK)],
                                 srcv.at[b], semS[b])
                pltpu.async_copy(dst_hbm.at[pl.ds(base + j * CHUNK, CHUNK)],
                                 dstv.at[b], semI[b])

            def gather_launch(j, b):
                pltpu.make_async_copy(
                    src_hbm.at[pl.ds(base + j * CHUNK, CHUNK)],
                    srcv.at[b], semS[b]).wait()
                pltpu.async_copy(data_hbm.at[srcv.at[b]], rows.at[b],
                                 semG[b])

            def drain(j, b):
                pltpu.make_async_copy(data_hbm.at[srcv.at[b]], rows.at[b],
                                      semG[b]).wait()
                pltpu.make_async_copy(
                    dst_hbm.at[pl.ds(base + j * CHUNK, CHUNK)],
                    dstv.at[b], semI[b]).wait()
                pltpu.sync_copy(rows.at[b], acc.at[dstv.at[b]], add=True)
                if with_deg:
                    one = jnp.full((16,), 1.0, jnp.float32)
                    for kk in range(CHUNK // 16):
                        d16 = dstv[b, pl.ds(kk * 16, 16)]
                        plsc.addupdate_scatter(hist, [d16], one)

            for b in range(depth):
                idx_issue(b, b)
            for b in range(depth - 1):
                gather_launch(b, b)

            @pl.loop(0, nchunk // depth)
            def _(jj):
                j0 = jj * depth
                for b in range(depth):
                    j = j0 + b
                    drain(j, b)
                    jn = j + depth

                    @pl.when(jn < nchunk)
                    def _():
                        idx_issue(jn, b)

                    jg = j + depth - 1
                    bg = (b + depth - 1) % depth

                    @pl.when(jg < nchunk)
                    def _():
                        gather_launch(jg, bg)

            for b in range(nchunk % depth):
                drain(nchunk - nchunk % depth + b, b)

        @pl.when(c == 0)
        def _():
            run(dataA)

        @pl.when(c == 1)
        def _():
            run(dataB)

        plsc.subcore_barrier()

        @pl.when(c == 0)
        def _():
            _copy_rows(s, acc, accA_o)

        @pl.when(c == 1)
        def _():
            _copy_rows(s, acc, accB_o)

        if with_deg:
            pltpu.sync_copy(hist, hist_o.at[w])

    return k


_agg_split_deg = _make_agg(featsplit=False, with_deg=True, depth=3)
_agg_feat = _make_agg(featsplit=True, with_deg=False, depth=4)
_agg_split = _make_agg(featsplit=False, with_deg=False, depth=4)

_BN = 1000


def _dot(a, b):
    return jnp.dot(a, b, preferred_element_type=jnp.float32)


def _mm(x, w):
    k, m = w.shape

    def body(x_ref, w_ref, o_ref):
        o_ref[...] = _dot(x_ref[...], w_ref[...])

    return pl.pallas_call(
        body,
        grid=(N // _BN,),
        in_specs=[pl.BlockSpec((_BN, k), lambda i: (i, 0)),
                  pl.BlockSpec((k, m), lambda i: (0, 0))],
        out_specs=pl.BlockSpec((_BN, m), lambda i: (i, 0)),
        out_shape=jax.ShapeDtypeStruct((N, m), jnp.float32),
    )(x, w)


def _stage2(accA, accB, recip, xx, w1l, w1r, w2l, b1):
    def body(aA, aB, rp, x_ref, w1l_ref, w1r_ref, w2l_ref, b1_ref,
             h1_o, t2A_o, t2B_o):
        mean1 = (aA[...] + aB[...]) * rp[...]
        pre = (_dot(mean1, w1l_ref[...]) + _dot(x_ref[...], w1r_ref[...])
               + b1_ref[...])
        h1 = jnp.maximum(pre, 0.0)
        h1_o[...] = h1
        t2 = _dot(h1, w2l_ref[...])
        t2A_o[...] = t2[:, :128]
        t2B_o[...] = t2[:, 128:]

    return pl.pallas_call(
        body,
        grid=(N // _BN,),
        in_specs=[pl.BlockSpec((_BN, 128), lambda i: (i, 0)),
                  pl.BlockSpec((_BN, 128), lambda i: (i, 0)),
                  pl.BlockSpec((_BN, 1), lambda i: (i, 0)),
                  pl.BlockSpec((_BN, 128), lambda i: (i, 0)),
                  pl.BlockSpec((128, 512), lambda i: (0, 0)),
                  pl.BlockSpec((128, 512), lambda i: (0, 0)),
                  pl.BlockSpec((512, 256), lambda i: (0, 0)),
                  pl.BlockSpec((1, 512), lambda i: (0, 0))],
        out_specs=[pl.BlockSpec((_BN, 512), lambda i: (i, 0)),
                   pl.BlockSpec((_BN, 128), lambda i: (i, 0)),
                   pl.BlockSpec((_BN, 128), lambda i: (i, 0))],
        out_shape=[jax.ShapeDtypeStruct((N, 512), jnp.float32),
                   jax.ShapeDtypeStruct((N, 128), jnp.float32),
                   jax.ShapeDtypeStruct((N, 128), jnp.float32)],
    )(accA, accB, recip, xx, w1l, w1r, w2l, b1)


def _stage4(accA, accB, recip, h1, w2r, w3l, b2):
    def body(aA, aB, rp, h1_ref, w2r_ref, w3l_ref, b2_ref, h2_o, t3_o):
        agg = jnp.concatenate([aA[...], aB[...]], axis=1)
        r2 = _dot(h1_ref[...], w2r_ref[...])
        h2 = jnp.maximum(agg * rp[...] + r2 + b2_ref[...], 0.0)
        h2_o[...] = h2
        t3_o[...] = _dot(h2, w3l_ref[...])

    return pl.pallas_call(
        body,
        grid=(N // _BN,),
        in_specs=[pl.BlockSpec((_BN, 128), lambda i: (i, 0)),
                  pl.BlockSpec((_BN, 128), lambda i: (i, 0)),
                  pl.BlockSpec((_BN, 1), lambda i: (i, 0)),
                  pl.BlockSpec((_BN, 512), lambda i: (i, 0)),
                  pl.BlockSpec((512, 256), lambda i: (0, 0)),
                  pl.BlockSpec((256, 128), lambda i: (0, 0)),
                  pl.BlockSpec((1, 256), lambda i: (0, 0))],
        out_specs=[pl.BlockSpec((_BN, 256), lambda i: (i, 0)),
                   pl.BlockSpec((_BN, 128), lambda i: (i, 0))],
        out_shape=[jax.ShapeDtypeStruct((N, 256), jnp.float32),
                   jax.ShapeDtypeStruct((N, 128), jnp.float32)],
    )(accA, accB, recip, h1, w2r, w3l, b2)


def _stage6(accA, accB, recip, h2, w3r, b3):
    def body(aA, aB, rp, h2_ref, w3r_ref, b3_ref, o_ref):
        o_ref[...] = ((aA[...] + aB[...]) * rp[...]
                      + _dot(h2_ref[...], w3r_ref[...]) + b3_ref[...])

    return pl.pallas_call(
        body,
        grid=(N // _BN,),
        in_specs=[pl.BlockSpec((_BN, 128), lambda i: (i, 0)),
                  pl.BlockSpec((_BN, 128), lambda i: (i, 0)),
                  pl.BlockSpec((_BN, 1), lambda i: (i, 0)),
                  pl.BlockSpec((_BN, 256), lambda i: (i, 0)),
                  pl.BlockSpec((256, 128), lambda i: (0, 0)),
                  pl.BlockSpec((1, 128), lambda i: (0, 0))],
        out_specs=pl.BlockSpec((_BN, 128), lambda i: (i, 0)),
        out_shape=jax.ShapeDtypeStruct((N, 128), jnp.float32),
    )(accA, accB, recip, h2, w3r, b3)


def kernel(x, edge_index, W1_l, b1, W1_r, W2_l, b2, W2_r, W3_l, b3, W3_r):
    src = edge_index[0]
    dst = edge_index[1]

    W1rT = W1_r.T
    W1lT = W1_l.T
    W2lT = W2_l.T
    W2rT = W2_r.T
    W3lTp = jnp.pad(W3_l, ((0, 121), (0, 0))).T
    W3rTp = jnp.pad(W3_r, ((0, 121), (0, 0))).T
    b1r = b1.reshape(1, 512)
    b2r = b2.reshape(1, 256)
    b3r = jnp.pad(b3, (0, 121)).reshape(1, 128)

    z128 = jnp.zeros((N, 128), jnp.float32)
    z1d = jnp.zeros((HR * 128,), jnp.float32)

    pA1, pB1, hists = _agg_split_deg(x, x, src, dst, z128, z1d)
    deg = hists.sum(axis=0)[:N]
    recip = (1.0 / jnp.maximum(deg, 1.0)).reshape(N, 1)
    h1, t2A, t2B = _stage2(pA1, pB1, recip, x, W1lT, W1rT, W2lT, b1r)

    accA2, accB2 = _agg_feat(t2A, t2B, src, dst, z128, z1d)
    h2, t3 = _stage4(accA2, accB2, recip, h1, W2rT, W3lTp, b2r)

    pA3, pB3 = _agg_split(t3, t3, src, dst, z128, z1d)
    outp = _stage6(pA3, pB3, recip, h2, W3rTp, b3r)
    return outp[:, :7]


# speedup vs baseline: 16.2920x; 1.0013x over previous
import dataclasses
import functools

import jax
import jax.numpy as jnp
from jax import lax
from jax.experimental import pallas as pl
from jax.experimental.pallas import tpu as pltpu
from jax.experimental.pallas import tpu_sc as plsc

N = 10000
E = 320000
NC = 2
NS = 16
NW = NC * NS
CHUNK = 80
RPT = 624
TAIL = N - NS * RPT
HR = 80

_MESH = plsc.VectorSubcoreMesh(core_axis_name="c", subcore_axis_name="s")

_SC_PARAMS = pltpu.CompilerParams()
if "needs_layout_passes" in pltpu.CompilerParams.__dataclass_fields__:
    _SC_PARAMS = dataclasses.replace(_SC_PARAMS, needs_layout_passes=False)


def _copy_rows(s, src, dst):
    pltpu.sync_copy(src.at[pl.ds(s * RPT, RPT)], dst.at[pl.ds(s * RPT, RPT)])

    @pl.when(s == NS - 1)
    def _():
        pltpu.sync_copy(src.at[pl.ds(NS * RPT, TAIL)],
                        dst.at[pl.ds(NS * RPT, TAIL)])


def _make_agg(featsplit, with_deg, depth):
    out_types = [jax.ShapeDtypeStruct((N, 128), jnp.float32),
                 jax.ShapeDtypeStruct((N, 128), jnp.float32)]
    ept = E // NS if featsplit else E // NW
    nchunk = ept // CHUNK
    scratch = [
        pltpu.VMEM((depth, CHUNK), jnp.int32),
        pltpu.VMEM((2 * depth, CHUNK), jnp.int32),
        pltpu.VMEM((depth, CHUNK, 128), jnp.float32),
        pltpu.VMEM_SHARED((N, 128), jnp.float32),
    ] + [pltpu.SemaphoreType.DMA] * (5 * depth)
    if with_deg:
        out_types.append(jax.ShapeDtypeStruct((NW, HR * 128), jnp.float32))
        scratch.append(pltpu.VMEM((HR * 128,), jnp.float32))

    @functools.partial(pl.kernel, out_type=tuple(out_types), mesh=_MESH,
                       scratch_types=scratch, compiler_params=_SC_PARAMS)
    def k(dataA, dataB, src_hbm, dst_hbm, z128, z1d_hbm, *refs):
        if with_deg:
            (accA_o, accB_o, hist_o, srcv, dstv, rows, acc,
             *sems, hist) = refs
        else:
            accA_o, accB_o, srcv, dstv, rows, acc, *sems = refs
        R = 2 * depth
        semG = sems[:depth]
        semS = sems[depth:2 * depth]
        semA = sems[2 * depth:3 * depth]
        semI = sems[3 * depth:3 * depth + R]
        c = lax.axis_index("c")
        s = lax.axis_index("s")
        w = c * NS + s
        base = (s if featsplit else w) * ept

        _copy_rows(s, z128, acc)
        if with_deg:
            pltpu.sync_copy(z1d_hbm, hist)
        plsc.subcore_barrier()

        def run(data_hbm):
            def idx_issue(j, u):
                b = u % depth
                pltpu.async_copy(src_hbm.at[pl.ds(base + j * CHUNK, CHUNK)],
                                 srcv.at[b], semS[b])
                pltpu.async_copy(dst_hbm.at[pl.ds(base + j * CHUNK, CHUNK)],
                                 dstv.at[u], semI[u])

            def scatter_wait(b, u):
                pltpu.make_async_copy(rows.at[b], acc.at[dstv.at[u]],
                                      semA[b]).wait()

            def gather_launch(j, u, wait_scatter):
                b = u % depth
                pltpu.make_async_copy(
                    src_hbm.at[pl.ds(base + j * CHUNK, CHUNK)],
                    srcv.at[b], semS[b]).wait()
                if wait_scatter == "traced":
                    @pl.when(j >= depth)
                    def _():
                        scatter_wait(b, (u + depth) % R)
                elif wait_scatter:
                    scatter_wait(b, (u + depth) % R)
                pltpu.async_copy(data_hbm.at[srcv.at[b]], rows.at[b],
                                 semG[b])

            def drain(j, u):
                b = u % depth
                pltpu.make_async_copy(data_hbm.at[srcv.at[b]], rows.at[b],
                                      semG[b]).wait()
                pltpu.make_async_copy(
                    dst_hbm.at[pl.ds(base + j * CHUNK, CHUNK)],
                    dstv.at[u], semI[u]).wait()
                pltpu.async_copy(rows.at[b], acc.at[dstv.at[u]],
                                 semA[b], add=True)
                if with_deg:
                    one = jnp.full((16,), 1.0, jnp.float32)
                    for kk in range(CHUNK // 16):
                        d16 = dstv[u, pl.ds(kk * 16, 16)]
                        plsc.addupdate_scatter(hist, [d16], one)

            for u in range(depth):
                idx_issue(u, u)
            for u in range(depth - 1):
                gather_launch(u, u, wait_scatter=False)

            main = (nchunk // R) * R

            @pl.loop(0, nchunk // R)
            def _(jj):
                j0 = jj * R
                for u in range(R):
                    j = j0 + u
                    drain(j, u)
                    jn = j + depth

                    @pl.when(jn < nchunk)
                    def _():
                        idx_issue(jn, (u + depth) % R)

                    jg = j + depth - 1

                    @pl.when(jg < nchunk)
                    def _():
                        gather_launch(jg, (u + depth - 1) % R, "traced")

            for i in range(nchunk % R):
                j = main + i
                drain(j, i)
                jn = j + depth
                if jn < nchunk:
                    idx_issue(jn, (i + depth) % R)
                jg = j + depth - 1
                if jg < nchunk:
                    gather_launch(jg, (i + depth - 1) % R, jg >= depth)
            for j in range(nchunk - depth, nchunk):
                scatter_wait(j % depth, j % R)

        @pl.when(c == 0)
        def _():
            run(dataA)

        @pl.when(c == 1)
        def _():
            run(dataB)

        plsc.subcore_barrier()

        @pl.when(c == 0)
        def _():
            _copy_rows(s, acc, accA_o)

        @pl.when(c == 1)
        def _():
            _copy_rows(s, acc, accB_o)

        if with_deg:
            pltpu.sync_copy(hist, hist_o.at[w])

    return k


_agg_split_deg = _make_agg(featsplit=False, with_deg=True, depth=3)
_agg_feat = _make_agg(featsplit=True, with_deg=False, depth=4)
_agg_split = _make_agg(featsplit=False, with_deg=False, depth=4)

_BN = 1000


def _dot(a, b):
    return jnp.dot(a, b, preferred_element_type=jnp.float32)


def _stage2(accA, accB, recip, xx, w1l, w1r, w2l, b1):
    def body(aA, aB, rp, x_ref, w1l_ref, w1r_ref, w2l_ref, b1_ref,
             h1_o, t2A_o, t2B_o):
        mean1 = (aA[...] + aB[...]) * rp[...]
        pre = (_dot(mean1, w1l_ref[...]) + _dot(x_ref[...], w1r_ref[...])
               + b1_ref[...])
        h1 = jnp.maximum(pre, 0.0)
        h1_o[...] = h1
        t2 = _dot(h1, w2l_ref[...])
        t2A_o[...] = t2[:, :128]
        t2B_o[...] = t2[:, 128:]

    return pl.pallas_call(
        body,
        grid=(N // _BN,),
        in_specs=[pl.BlockSpec((_BN, 128), lambda i: (i, 0)),
                  pl.BlockSpec((_BN, 128), lambda i: (i, 0)),
                  pl.BlockSpec((_BN, 1), lambda i: (i, 0)),
                  pl.BlockSpec((_BN, 128), lambda i: (i, 0)),
                  pl.BlockSpec((128, 512), lambda i: (0, 0)),
                  pl.BlockSpec((128, 512), lambda i: (0, 0)),
                  pl.BlockSpec((512, 256), lambda i: (0, 0)),
                  pl.BlockSpec((1, 512), lambda i: (0, 0))],
        out_specs=[pl.BlockSpec((_BN, 512), lambda i: (i, 0)),
                   pl.BlockSpec((_BN, 128), lambda i: (i, 0)),
                   pl.BlockSpec((_BN, 128), lambda i: (i, 0))],
        out_shape=[jax.ShapeDtypeStruct((N, 512), jnp.float32),
                   jax.ShapeDtypeStruct((N, 128), jnp.float32),
                   jax.ShapeDtypeStruct((N, 128), jnp.float32)],
    )(accA, accB, recip, xx, w1l, w1r, w2l, b1)


def _stage4(accA, accB, recip, h1, w2r, w3l, b2):
    def body(aA, aB, rp, h1_ref, w2r_ref, w3l_ref, b2_ref, h2_o, t3_o):
        agg = jnp.concatenate([aA[...], aB[...]], axis=1)
        r2 = _dot(h1_ref[...], w2r_ref[...])
        h2 = jnp.maximum(agg * rp[...] + r2 + b2_ref[...], 0.0)
        h2_o[...] = h2
        t3_o[...] = _dot(h2, w3l_ref[...])

    return pl.pallas_call(
        body,
        grid=(N // _BN,),
        in_specs=[pl.BlockSpec((_BN, 128), lambda i: (i, 0)),
                  pl.BlockSpec((_BN, 128), lambda i: (i, 0)),
                  pl.BlockSpec((_BN, 1), lambda i: (i, 0)),
                  pl.BlockSpec((_BN, 512), lambda i: (i, 0)),
                  pl.BlockSpec((512, 256), lambda i: (0, 0)),
                  pl.BlockSpec((256, 128), lambda i: (0, 0)),
                  pl.BlockSpec((1, 256), lambda i: (0, 0))],
        out_specs=[pl.BlockSpec((_BN, 256), lambda i: (i, 0)),
                   pl.BlockSpec((_BN, 128), lambda i: (i, 0))],
        out_shape=[jax.ShapeDtypeStruct((N, 256), jnp.float32),
                   jax.ShapeDtypeStruct((N, 128), jnp.float32)],
    )(accA, accB, recip, h1, w2r, w3l, b2)


def _stage6(accA, accB, recip, h2, w3r, b3):
    def body(aA, aB, rp, h2_ref, w3r_ref, b3_ref, o_ref):
        o_ref[...] = ((aA[...] + aB[...]) * rp[...]
                      + _dot(h2_ref[...], w3r_ref[...]) + b3_ref[...])

    return pl.pallas_call(
        body,
        grid=(N // _BN,),
        in_specs=[pl.BlockSpec((_BN, 128), lambda i: (i, 0)),
                  pl.BlockSpec((_BN, 128), lambda i: (i, 0)),
                  pl.BlockSpec((_BN, 1), lambda i: (i, 0)),
                  pl.BlockSpec((_BN, 256), lambda i: (i, 0)),
                  pl.BlockSpec((256, 128), lambda i: (0, 0)),
                  pl.BlockSpec((1, 128), lambda i: (0, 0))],
        out_specs=pl.BlockSpec((_BN, 128), lambda i: (i, 0)),
        out_shape=jax.ShapeDtypeStruct((N, 128), jnp.float32),
    )(accA, accB, recip, h2, w3r, b3)


def kernel(x, edge_index, W1_l, b1, W1_r, W2_l, b2, W2_r, W3_l, b3, W3_r):
    src = edge_index[0]
    dst = edge_index[1]

    W1rT = W1_r.T
    W1lT = W1_l.T
    W2lT = W2_l.T
    W2rT = W2_r.T
    W3lTp = jnp.pad(W3_l, ((0, 121), (0, 0))).T
    W3rTp = jnp.pad(W3_r, ((0, 121), (0, 0))).T
    b1r = b1.reshape(1, 512)
    b2r = b2.reshape(1, 256)
    b3r = jnp.pad(b3, (0, 121)).reshape(1, 128)

    z128 = jnp.zeros((N, 128), jnp.float32)
    z1d = jnp.zeros((HR * 128,), jnp.float32)

    pA1, pB1, hists = _agg_split_deg(x, x, src, dst, z128, z1d)
    deg = hists.sum(axis=0)[:N]
    recip = (1.0 / jnp.maximum(deg, 1.0)).reshape(N, 1)
    h1, t2A, t2B = _stage2(pA1, pB1, recip, x, W1lT, W1rT, W2lT, b1r)

    accA2, accB2 = _agg_feat(t2A, t2B, src, dst, z128, z1d)
    h2, t3 = _stage4(accA2, accB2, recip, h1, W2rT, W3lTp, b2r)

    pA3, pB3 = _agg_split(t3, t3, src, dst, z128, z1d)
    outp = _stage6(pA3, pB3, recip, h2, W3rTp, b3r)
    return outp[:, :7]
